# baseline (speedup 1.0000x reference)
import functools

import jax
import jax.numpy as jnp
from jax import lax
from jax.experimental import pallas as pl
from jax.experimental.pallas import tpu as pltpu
from jax.experimental.pallas import tpu_sc as plsc

NC = 2
NS = 16
NW = NC * NS
K = 80
L = 16


@functools.lru_cache(maxsize=None)
def _sc_segsum(n, e, d, with_counts):
    epw = e // NW
    nchunk = epw // K
    rquot = 8 * (n // (NS * 8))
    tail = n - NS * rquot
    assert epw % K == 0 and tail % 8 == 0 and tail <= rquot and n % L == 0

    mesh = plsc.VectorSubcoreMesh(core_axis_name="c", subcore_axis_name="s")
    out_type = [jax.ShapeDtypeStruct((NC, n, d), jnp.float32)]
    scratch = {
        "src_c": pltpu.VMEM((2, K), jnp.int32),
        "dst_b": pltpu.VMEM((2, K), jnp.int32),
        "rows0": pltpu.VMEM((K, d), jnp.float32),
        "rows1": pltpu.VMEM((K, d), jnp.float32),
        "acc_s": pltpu.VMEM_SHARED((n, d), jnp.float32),
        "gsem0": pltpu.SemaphoreType.DMA,
        "gsem1": pltpu.SemaphoreType.DMA,
        "ssem0": pltpu.SemaphoreType.DMA,
        "ssem1": pltpu.SemaphoreType.DMA,
        "isem0": pltpu.SemaphoreType.DMA,
        "isem1": pltpu.SemaphoreType.DMA,
    }
    if with_counts:
        out_type.append(jax.ShapeDtypeStruct((NW * n,), jnp.float32))
        scratch["cnt_v"] = pltpu.VMEM((n,), jnp.float32)

    def body(x_hbm, src_hbm, dst_hbm, zeros_hbm, sums_hbm, cnts_hbm=None,
             *, src_c, dst_b, rows0, rows1, acc_s, gsem0, gsem1, ssem0,
             ssem1, isem0, isem1, cnt_v=None):
        c = lax.axis_index("c")
        s = lax.axis_index("s")
        w = c * NS + s
        rbase = s * rquot
        ebase = w * epw

        def over_rows(fn):
            fn(rbase, rquot)
            if tail:
                @pl.when(s == NS - 1)
                def _():
                    fn(NS * rquot, tail)

        over_rows(lambda b, m: pltpu.sync_copy(
            zeros_hbm.at[pl.ds(0, m)], acc_s.at[pl.ds(b, m)]))
        if with_counts:
            def zero_cnt(i, carry):
                cnt_v[pl.ds(i * L, L)] = jnp.zeros((L,), jnp.float32)
                return carry
            lax.fori_loop(0, n // L, zero_cnt, 0)
        plsc.subcore_barrier()

        ones16 = jnp.ones((L,), jnp.float32)
        isems = (isem0, isem1)

        def idx_load(i, p):
            off = ebase + i * K
            sem = isems[p]
            pltpu.async_copy(src_hbm.at[pl.ds(off, K)], src_c.at[p], sem)
            pltpu.async_copy(dst_hbm.at[pl.ds(off, K)], dst_b.at[p], sem)

        def iwait(i, p):
            off = ebase + i * K
            sem = isems[p]
            pltpu.make_async_copy(src_hbm.at[pl.ds(off, K)], src_c.at[p],
                                  sem).wait()
            pltpu.make_async_copy(dst_hbm.at[pl.ds(off, K)], dst_b.at[p],
                                  sem).wait()

        def gather(p, buf, sem):
            return pltpu.async_copy(x_hbm.at[src_c.at[p]], buf, sem)

        def gwait(p, buf, sem):
            pltpu.make_async_copy(x_hbm.at[src_c.at[p]], buf, sem).wait()

        def scatter(p, buf, sem):
            return pltpu.async_copy(buf, acc_s.at[dst_b.at[p]], sem,
                                    add=True)

        def swait(p, buf, sem):
            pltpu.make_async_copy(buf, acc_s.at[dst_b.at[p]], sem).wait()

        def counts(p):
            if with_counts:
                for j in range(K // L):
                    idx = dst_b[p, pl.ds(j * L, L)]
                    plsc.addupdate_scatter(cnt_v, [idx], ones16)

        assert nchunk % 2 == 1 and nchunk >= 3
        npair = nchunk // 2
        idx_load(0, 0)
        iwait(0, 0)
        gather(0, rows0, gsem0)

        def pair(t, carry):
            i0 = 2 * t

            @pl.when(t > 0)
            def _():
                swait(1, rows1, ssem1)
            idx_load(i0 + 1, 1)
            counts(0)
            iwait(i0 + 1, 1)
            gather(1, rows1, gsem1)
            gwait(0, rows0, gsem0)
            scatter(0, rows0, ssem0).wait()
            idx_load(i0 + 2, 0)
            counts(1)
            iwait(i0 + 2, 0)
            gwait(1, rows1, gsem1)
            scatter(1, rows1, ssem1)
            gather(0, rows0, gsem0)
            return carry

        lax.fori_loop(0, npair, pair, 0)
        swait(1, rows1, ssem1)
        counts(0)
        gwait(0, rows0, gsem0)
        scatter(0, rows0, ssem0).wait()
        plsc.subcore_barrier()
        over_rows(lambda b, m: pltpu.sync_copy(
            acc_s.at[pl.ds(b, m)], sums_hbm.at[c].at[pl.ds(b, m)]))
        if with_counts:
            pltpu.sync_copy(cnt_v, cnts_hbm.at[pl.ds(w * n, n)])

    if with_counts:
        def body_wc(x, src, dst, z, sums, cnts, **scr):
            body(x, src, dst, z, sums, cnts, **scr)
        fn = body_wc
    else:
        def body_nc(x, src, dst, z, sums, **scr):
            body(x, src, dst, z, sums, None, **scr)
        fn = body_nc

    return pl.kernel(
        fn, out_type=out_type, mesh=mesh, scratch_types=scratch,
        compiler_params=pltpu.CompilerParams(needs_layout_passes=False))


def _mm_body(a_ref, w_ref, o_ref):
    o_ref[...] = jnp.dot(a_ref[...], w_ref[...],
                         preferred_element_type=jnp.float32)


def _tc1_body(sums_ref, cnts_ref, xwr_ref, wl_ref, bl_ref, g_ref,
              b_ref, o_ref, cnt_ref):
    cnt = jnp.maximum(jnp.sum(cnts_ref[...], axis=0), 1.0)[:, None]
    cnt_ref[...] = cnt
    ssum = sums_ref[0] + sums_ref[1]
    mean = ssum / cnt
    h = (jnp.dot(mean, wl_ref[...], preferred_element_type=jnp.float32)
         + bl_ref[...] + xwr_ref[...])
    mu = jnp.mean(h, axis=0, keepdims=True)
    var = jnp.mean((h - mu) ** 2, axis=0, keepdims=True)
    hn = (h - mu) * lax.rsqrt(var + 1e-5) * g_ref[...] + b_ref[...]
    o_ref[...] = jnp.where(hn >= 0, hn, 0.01 * hn)


def _tc2_body(sums_ref, cnt_ref, hwr_ref, wl_ref, bl_ref, o_ref):
    ssum = sums_ref[0] + sums_ref[1]
    mean = ssum / cnt_ref[...]
    o_ref[...] = (jnp.dot(mean, wl_ref[...], preferred_element_type=jnp.float32)
                  + bl_ref[...] + hwr_ref[...])


def kernel(x, edge_index, Wl1, bl1, Wr1, gamma, beta, Wl2, bl2, Wr2):
    n, d = x.shape
    e = edge_index.shape[1]
    src = edge_index[0].astype(jnp.int32)
    dst = edge_index[1].astype(jnp.int32)
    rquot = 8 * (n // (NS * 8))
    zeros = jnp.zeros((rquot, d), jnp.float32)
    mm = pl.pallas_call(
        _mm_body, out_shape=jax.ShapeDtypeStruct((n, d), jnp.float32))

    xwr = mm(x, Wr1)
    sums1, cnts = _sc_segsum(n, e, d, True)(x, src, dst, zeros)
    h, cnt_col = pl.pallas_call(
        _tc1_body,
        out_shape=[jax.ShapeDtypeStruct((n, d), jnp.float32),
                   jax.ShapeDtypeStruct((n, 1), jnp.float32)],
    )(sums1, cnts.reshape(NW, n), xwr, Wl1, bl1.reshape(1, -1),
      gamma.reshape(1, -1), beta.reshape(1, -1))
    hwr = mm(h, Wr2)
    (sums2,) = _sc_segsum(n, e, d, False)(h, src, dst, zeros)
    out = pl.pallas_call(
        _tc2_body,
        out_shape=jax.ShapeDtypeStruct((n, d), jnp.float32),
    )(sums2, cnt_col, hwr, Wl2, bl2.reshape(1, -1))
    return out


# speedup vs baseline: 1.3816x; 1.3816x over previous
import functools

import jax
import jax.numpy as jnp
from jax import lax
from jax.experimental import pallas as pl
from jax.experimental.pallas import tpu as pltpu
from jax.experimental.pallas import tpu_sc as plsc

NC = 2
NS = 16
NW = NC * NS
K = 80
L = 16


@functools.lru_cache(maxsize=None)
def _sc_segsum(n, e, d, with_counts):
    epw = e // NW
    nchunk = epw // K
    rquot = 8 * (n // (NS * 8))
    tail = n - NS * rquot
    assert epw % K == 0 and tail % 8 == 0 and tail <= rquot and n % L == 0

    mesh = plsc.VectorSubcoreMesh(core_axis_name="c", subcore_axis_name="s")
    out_type = [jax.ShapeDtypeStruct((NC, n, d), jnp.float32)]
    scratch = {
        "src_c": pltpu.VMEM((6, K), jnp.int32),
        "dst_b": pltpu.VMEM((6, K), jnp.int32),
        "rows0": pltpu.VMEM((K, d), jnp.float32),
        "rows1": pltpu.VMEM((K, d), jnp.float32),
        "rows2": pltpu.VMEM((K, d), jnp.float32),
        "acc_s": pltpu.VMEM_SHARED((n, d), jnp.float32),
        "gsem0": pltpu.SemaphoreType.DMA,
        "gsem1": pltpu.SemaphoreType.DMA,
        "gsem2": pltpu.SemaphoreType.DMA,
        "ssem0": pltpu.SemaphoreType.DMA,
        "ssem1": pltpu.SemaphoreType.DMA,
        "ssem2": pltpu.SemaphoreType.DMA,
        "isem0": pltpu.SemaphoreType.DMA,
        "isem1": pltpu.SemaphoreType.DMA,
        "isem2": pltpu.SemaphoreType.DMA,
        "isem3": pltpu.SemaphoreType.DMA,
        "isem4": pltpu.SemaphoreType.DMA,
        "isem5": pltpu.SemaphoreType.DMA,
    }
    if with_counts:
        out_type.append(jax.ShapeDtypeStruct((NW * n,), jnp.float32))
        scratch["cnt_v"] = pltpu.VMEM((n,), jnp.float32)

    def body(x_hbm, src_hbm, dst_hbm, zeros_hbm, sums_hbm, cnts_hbm=None,
             *, src_c, dst_b, rows0, rows1, rows2, acc_s, gsem0, gsem1,
             gsem2, ssem0, ssem1, ssem2, isem0, isem1, isem2, isem3,
             isem4, isem5, cnt_v=None):
        c = lax.axis_index("c")
        s = lax.axis_index("s")
        w = c * NS + s
        rbase = s * rquot
        ebase = w * epw

        def over_rows(fn):
            fn(rbase, rquot)
            if tail:
                @pl.when(s == NS - 1)
                def _():
                    fn(NS * rquot, tail)

        over_rows(lambda b, m: pltpu.sync_copy(
            zeros_hbm.at[pl.ds(0, m)], acc_s.at[pl.ds(b, m)]))
        if with_counts:
            def zero_cnt(i, carry):
                cnt_v[pl.ds(i * L, L)] = jnp.zeros((L,), jnp.float32)
                return carry
            lax.fori_loop(0, n // L, zero_cnt, 0)
        plsc.subcore_barrier()

        ones16 = jnp.ones((L,), jnp.float32)
        rows = (rows0, rows1, rows2)
        gsems = (gsem0, gsem1, gsem2)
        ssems = (ssem0, ssem1, ssem2)
        isems = (isem0, isem1, isem2, isem3, isem4, isem5)

        def idx_load(i, q):
            off = ebase + i * K
            pltpu.async_copy(src_hbm.at[pl.ds(off, K)], src_c.at[q],
                             isems[q])
            pltpu.async_copy(dst_hbm.at[pl.ds(off, K)], dst_b.at[q],
                             isems[q])

        def iwait(i, q):
            off = ebase + i * K
            pltpu.make_async_copy(src_hbm.at[pl.ds(off, K)], src_c.at[q],
                                  isems[q]).wait()
            pltpu.make_async_copy(dst_hbm.at[pl.ds(off, K)], dst_b.at[q],
                                  isems[q]).wait()

        def gfire(p, q):
            pltpu.async_copy(x_hbm.at[src_c.at[q]], rows[p], gsems[p])

        def gwait(p, q):
            pltpu.make_async_copy(x_hbm.at[src_c.at[q]], rows[p],
                                  gsems[p]).wait()

        def sfire(p, q):
            pltpu.async_copy(rows[p], acc_s.at[dst_b.at[q]], ssems[p],
                             add=True)

        def swaitf(p, q):
            pltpu.make_async_copy(rows[p], acc_s.at[dst_b.at[q]],
                                  ssems[p]).wait()

        def counts(q):
            if with_counts:
                for j in range(K // L):
                    idx = dst_b[q, pl.ds(j * L, L)]
                    plsc.addupdate_scatter(cnt_v, [idx], ones16)

        def substep(i, k, head=False, fire_g=True, fire_i=True):
            p, pn = k % 3, (k + 1) % 3
            q, qn, q2 = k % 6, (k + 1) % 6, (k + 2) % 6
            if not head:
                swaitf((k + 1) % 3, (k + 4) % 6)
            if fire_g:
                iwait(i + 1, qn)
                gfire(pn, qn)
            if fire_i:
                idx_load(i + 2, q2)
            gwait(p, q)
            sfire(p, q)
            counts(q)

        assert nchunk == 125
        idx_load(0, 0)
        iwait(0, 0)
        gfire(0, 0)
        idx_load(1, 1)
        substep(0, 0, head=True)
        substep(1, 1, head=True)

        def six(t, carry):
            i0 = 6 * t + 2
            for k in range(6):
                substep(i0 + k, 2 + k)
            return carry

        lax.fori_loop(0, 19, six, 0)
        for i in range(116, 125):
            substep(i, i % 6, fire_g=(i + 1 < 125), fire_i=(i + 2 < 125))
        swaitf(123 % 3, 123 % 6)
        swaitf(124 % 3, 124 % 6)
        plsc.subcore_barrier()
        over_rows(lambda b, m: pltpu.sync_copy(
            acc_s.at[pl.ds(b, m)], sums_hbm.at[c].at[pl.ds(b, m)]))
        if with_counts:
            pltpu.sync_copy(cnt_v, cnts_hbm.at[pl.ds(w * n, n)])

    if with_counts:
        def body_wc(x, src, dst, z, sums, cnts, **scr):
            body(x, src, dst, z, sums, cnts, **scr)
        fn = body_wc
    else:
        def body_nc(x, src, dst, z, sums, **scr):
            body(x, src, dst, z, sums, None, **scr)
        fn = body_nc

    return pl.kernel(
        fn, out_type=out_type, mesh=mesh, scratch_types=scratch,
        compiler_params=pltpu.CompilerParams(needs_layout_passes=False))


def _tc1_body(sums_ref, cnts_ref, x_ref, wl_ref, bl_ref, wr_ref, g_ref,
              b_ref, o_ref, cnt_ref):
    cnt = jnp.maximum(jnp.sum(cnts_ref[...], axis=0), 1.0)[:, None]
    cnt_ref[...] = cnt
    ssum = sums_ref[0] + sums_ref[1]
    mean = ssum / cnt
    h = (jnp.dot(mean, wl_ref[...], preferred_element_type=jnp.float32)
         + bl_ref[...]
         + jnp.dot(x_ref[...], wr_ref[...], preferred_element_type=jnp.float32))
    mu = jnp.mean(h, axis=0, keepdims=True)
    var = jnp.mean((h - mu) ** 2, axis=0, keepdims=True)
    hn = (h - mu) * lax.rsqrt(var + 1e-5) * g_ref[...] + b_ref[...]
    o_ref[...] = jnp.where(hn >= 0, hn, 0.01 * hn)


def _tc2_body(sums_ref, cnt_ref, h_ref, wl_ref, bl_ref, wr_ref, o_ref):
    ssum = sums_ref[0] + sums_ref[1]
    mean = ssum / cnt_ref[...]
    o_ref[...] = (jnp.dot(mean, wl_ref[...], preferred_element_type=jnp.float32)
                  + bl_ref[...]
                  + jnp.dot(h_ref[...], wr_ref[...],
                            preferred_element_type=jnp.float32))


def kernel(x, edge_index, Wl1, bl1, Wr1, gamma, beta, Wl2, bl2, Wr2):
    n, d = x.shape
    e = edge_index.shape[1]
    src = edge_index[0].astype(jnp.int32)
    dst = edge_index[1].astype(jnp.int32)
    rquot = 8 * (n // (NS * 8))
    zeros = jnp.zeros((rquot, d), jnp.float32)

    sums1, cnts = _sc_segsum(n, e, d, True)(x, src, dst, zeros)
    h, cnt_col = pl.pallas_call(
        _tc1_body,
        out_shape=[jax.ShapeDtypeStruct((n, d), jnp.float32),
                   jax.ShapeDtypeStruct((n, 1), jnp.float32)],
    )(sums1, cnts.reshape(NW, n), x, Wl1, bl1.reshape(1, -1), Wr1,
      gamma.reshape(1, -1), beta.reshape(1, -1))
    (sums2,) = _sc_segsum(n, e, d, False)(h, src, dst, zeros)
    out = pl.pallas_call(
        _tc2_body,
        out_shape=jax.ShapeDtypeStruct((n, d), jnp.float32),
    )(sums2, cnt_col, h, Wl2, bl2.reshape(1, -1), Wr2)
    return out
